# baseline (speedup 1.0000x reference)
import functools

import jax
import jax.numpy as jnp
from jax import lax
from jax.experimental import pallas as pl
from jax.experimental.pallas import tpu as pltpu
from jax.experimental.pallas import tpu_sc as plsc

POSITIVE_OVERLAP = 0.1
POSITIVE_RADIUS = 0.1

LMAP = 4096
MAP_SIZE = LMAP * LMAP
PAD = 16384
MAP_N = MAP_SIZE + PAD
HALF = MAP_SIZE // 2
NC = 262144
KQ = 131072
NP = 262144
NCORES = 2
NSUB = 16
NTILES = NCORES * NSUB

GT_ROWS = NC // 128
Q_ROWS = KQ // 128
GT_ROWS_PER_TILE = GT_ROWS // NSUB
Q_ROWS_PER_TILE = Q_ROWS // NTILES
ZWORDS = 16384
ZDMA = HALF // NSUB // ZWORDS

PM_BLK = 8192
PM_GRID = NP // PM_BLK

_MESH = plsc.VectorSubcoreMesh(
    core_axis_name="c", subcore_axis_name="s", num_cores=NCORES,
    num_subcores=NSUB)

_F32 = jnp.float32
_I32 = jnp.int32


@functools.partial(
    pl.kernel,
    out_type=jax.ShapeDtypeStruct((MAP_N,), _F32),
    mesh=_MESH,
    scratch_types=[
        pltpu.VMEM((ZWORDS,), _F32),
        pltpu.VMEM((128,), _F32),
        pltpu.VMEM((GT_ROWS_PER_TILE, 128), _I32),
        pltpu.VMEM((GT_ROWS_PER_TILE, 128), _I32),
        pltpu.VMEM((GT_ROWS_PER_TILE, 128), _F32),
        pltpu.VMEM((GT_ROWS_PER_TILE, 128), _I32),
        pltpu.SemaphoreType.DMA,
        pltpu.SemaphoreType.DMA,
    ],
)
def _scatter_kernel(gtr, gts, ovl, zin, onein, map_out,
                    zbuf, ones_v, gtr_v, gts_v, ovl_v, idx_v, zsem, ssem):
    c = lax.axis_index("c")
    s = lax.axis_index("s")

    pltpu.sync_copy(zin, zbuf)
    zbase = c * HALF + s * (HALF // NSUB)
    zhandles = []
    for k in range(ZDMA):
        off = pl.multiple_of(zbase + k * ZWORDS, 8)
        zhandles.append(
            pltpu.async_copy(zbuf, map_out.at[pl.ds(off, ZWORDS)], zsem))

    pltpu.sync_copy(onein, ones_v)
    r0 = s * GT_ROWS_PER_TILE
    pltpu.sync_copy(gtr.at[pl.ds(r0, GT_ROWS_PER_TILE)], gtr_v)
    pltpu.sync_copy(gts.at[pl.ds(r0, GT_ROWS_PER_TILE)], gts_v)
    pltpu.sync_copy(ovl.at[pl.ds(r0, GT_ROWS_PER_TILE)], ovl_v)
    lo = c * HALF
    hi = lo + HALF
    lane = lax.iota(_I32, 16)

    def gt_row(j, _):
        def gt_chunk(q, _):
            sl = pl.ds(q * 16, 16)
            flat = gtr_v[j, sl] * LMAP + gts_v[j, sl]
            ok = ((ovl_v[j, sl] > POSITIVE_OVERLAP)
                  & (flat >= lo) & (flat < hi))
            dump = MAP_SIZE + j * 128 + q * 16 + lane
            idx_v[j, sl] = jnp.where(ok, flat, dump)
            return 0
        return lax.fori_loop(0, 8, gt_chunk, 0)

    lax.fori_loop(0, GT_ROWS_PER_TILE, gt_row, 0)

    for h in zhandles:
        h.wait()
    plsc.subcore_barrier()

    shandles = [
        pltpu.async_copy(ones_v, map_out.at[idx_v.at[j]], ssem)
        for j in range(GT_ROWS_PER_TILE)
    ]
    for h in shandles:
        h.wait()


@functools.partial(
    pl.kernel,
    out_type=jax.ShapeDtypeStruct((NTILES, 16), _F32),
    mesh=_MESH,
    scratch_types=[
        pltpu.VMEM((Q_ROWS_PER_TILE, 128), _I32),
        pltpu.VMEM((Q_ROWS_PER_TILE, 128), _I32),
        pltpu.VMEM((Q_ROWS_PER_TILE, 128), _I32),
        pltpu.VMEM((Q_ROWS_PER_TILE, 128), _F32),
        pltpu.VMEM((16,), _F32),
        pltpu.SemaphoreType.DMA,
    ],
)
def _gather_kernel(qr, qs, map_in, cp_out, qr_v, qs_v, qidx_v, gv_v, acc_v,
                   gsem):
    c = lax.axis_index("c")
    s = lax.axis_index("s")
    wid = s * NCORES + c
    r0 = wid * Q_ROWS_PER_TILE
    pltpu.sync_copy(qr.at[pl.ds(r0, Q_ROWS_PER_TILE)], qr_v)
    pltpu.sync_copy(qs.at[pl.ds(r0, Q_ROWS_PER_TILE)], qs_v)

    def q_row(j, _):
        def q_chunk(q, _):
            sl = pl.ds(q * 16, 16)
            qidx_v[j, sl] = qr_v[j, sl] * LMAP + qs_v[j, sl]
            return 0
        return lax.fori_loop(0, 8, q_chunk, 0)

    lax.fori_loop(0, Q_ROWS_PER_TILE, q_row, 0)
    ghandles = [
        pltpu.async_copy(map_in.at[qidx_v.at[j]], gv_v.at[j], gsem)
        for j in range(Q_ROWS_PER_TILE)
    ]
    for h in ghandles:
        h.wait()

    def s_row(j, acc):
        def s_chunk(q, acc):
            return acc + gv_v[j, pl.ds(q * 16, 16)]
        return lax.fori_loop(0, 8, s_chunk, acc)

    acc = lax.fori_loop(0, Q_ROWS_PER_TILE, s_row, jnp.zeros((16,), _F32))
    acc_v[...] = acc
    pltpu.sync_copy(acc_v, cp_out.at[wid])


def _pm_body(s_blk, r_blk, rt_ref, t_ref, out_ref):
    st = jnp.dot(s_blk[...], rt_ref[...], preferred_element_type=_F32)
    st = st + t_ref[...]
    d = r_blk[...] - st
    nrm = jnp.sqrt(jnp.sum(d * d, axis=1))
    cnt = jnp.sum((nrm < POSITIVE_RADIUS).astype(_F32))
    out_ref[...] = cnt.reshape(1, 1, 1)


_pm_kernel = pl.pallas_call(
    _pm_body,
    out_shape=jax.ShapeDtypeStruct((PM_GRID, 1, 1), _F32),
    grid=(PM_GRID,),
    in_specs=[
        pl.BlockSpec((PM_BLK, 3), lambda i: (i, 0)),
        pl.BlockSpec((PM_BLK, 3), lambda i: (i, 0)),
        pl.BlockSpec((3, 3), lambda i: (0, 0)),
        pl.BlockSpec((1, 3), lambda i: (0, 0)),
    ],
    out_specs=pl.BlockSpec((1, 1, 1), lambda i: (i, 0, 0)),
)


def kernel(gt_node_corr_overlaps, gt_node_corr_indices, ref_node_corr_indices,
           src_node_corr_indices, transform, ref_corr_points, src_corr_points,
           estimated_transform, ref_length_c, src_length_c):
    gtr = (gt_node_corr_indices[:, 0] % ref_length_c).astype(_I32)
    gts = (gt_node_corr_indices[:, 1] % src_length_c).astype(_I32)
    gtr2 = gtr.reshape(GT_ROWS, 128)
    gts2 = gts.reshape(GT_ROWS, 128)
    ovl2 = gt_node_corr_overlaps.reshape(GT_ROWS, 128)
    qr2 = (ref_node_corr_indices % ref_length_c).astype(_I32).reshape(Q_ROWS, 128)
    qs2 = (src_node_corr_indices % src_length_c).astype(_I32).reshape(Q_ROWS, 128)
    zin = jnp.zeros((ZWORDS,), _F32)
    onein = jnp.ones((128,), _F32)

    corr_map = _scatter_kernel(gtr2, gts2, ovl2, zin, onein)
    cp_part = _gather_kernel(qr2, qs2, corr_map)

    rt = transform[:3, :3].T
    tv = transform[:3, 3].reshape(1, 3)
    pm_part = _pm_kernel(src_corr_points, ref_corr_points, rt, tv)

    c_precision = jnp.sum(cp_part) / jnp.float32(KQ)
    f_precision = jnp.sum(pm_part) / jnp.float32(NP)

    Rg, tg = transform[:3, :3], transform[:3, 3]
    Re, te = estimated_transform[:3, :3], estimated_transform[:3, 3]
    x = (jnp.trace(Rg.T @ Re) - 1.0) * 0.5
    rre = jnp.degrees(jnp.arccos(jnp.clip(x, -0.999999, 0.999999)))
    rte = jnp.linalg.norm(tg - te)

    return (c_precision, f_precision, rre, rte)
